# baseline (speedup 1.0000x reference)
import functools

import jax
import jax.numpy as jnp
from jax import lax
from jax.experimental import pallas as pl
from jax.experimental.pallas import tpu as pltpu
from jax.experimental.pallas import tpu_sc as plsc

NUM_NODES = 100000
FEAT_DIM = 128
EMB_DIM = 64
BATCH = 16384
OUT_DIM = FEAT_DIM + EMB_DIM

NC = 2
NS = 16
NW = NC * NS
BPW = BATCH // NW
NCHUNK = 4
C = BPW // NCHUNK
AROWS = (C * OUT_DIM) // FEAT_DIM
OUT_ROWS = (BATCH * OUT_DIM) // FEAT_DIM
L = 16

_mesh = plsc.VectorSubcoreMesh(core_axis_name="c", subcore_axis_name="s")


@functools.partial(
    pl.kernel,
    mesh=_mesh,
    out_type=jax.ShapeDtypeStruct((OUT_ROWS, FEAT_DIM), jnp.float32),
    scratch_types=[
        pltpu.VMEM((BPW + L,), jnp.int32),
        pltpu.VMEM((BPW,), jnp.int32),
        pltpu.VMEM((C, FEAT_DIM), jnp.float32),
        pltpu.VMEM((C, FEAT_DIM), jnp.float32),
        pltpu.VMEM((AROWS, FEAT_DIM), jnp.float32),
        pltpu.SemaphoreType.DMA,
    ],
    compiler_params=pltpu.CompilerParams(use_tc_tiling_on_sc=False),
)
def _encoder(idx_hbm, feat_hbm, emb2_hbm, out_hbm, idx_v, ix2_v, fbuf, ebuf,
             abuf, sem):
    wid = lax.axis_index("s") * NC + lax.axis_index("c")
    base = wid * BPW
    pltpu.sync_copy(idx_hbm.at[pl.ds(base, BPW)], idx_v.at[pl.ds(0, BPW)])

    def pair_ix(i, _):
        v = idx_v[pl.ds(i * L, L)]
        ix2_v[pl.ds(i * L, L)] = lax.shift_right_logical(v, 1)
        return _

    lax.fori_loop(0, BPW // L, pair_ix, 0, unroll=4)

    for j in range(NCHUNK):
        cp_f = pltpu.async_copy(
            feat_hbm.at[idx_v.at[pl.ds(j * C, C)]], fbuf, sem)
        cp_e = pltpu.async_copy(
            emb2_hbm.at[ix2_v.at[pl.ds(j * C, C)]], ebuf, sem)
        cp_f.wait()
        cp_e.wait()

        def assemble(k, _):
            r0 = 2 * k
            r1 = 2 * k + 1
            a0 = 3 * k
            iv = idx_v[pl.ds(j * C + r0, L)]
            p0 = pl.multiple_of((iv[0] & 1) * EMB_DIM, L)
            p1 = pl.multiple_of((iv[1] & 1) * EMB_DIM, L)
            for c in range(8):
                abuf[a0, pl.ds(c * L, L)] = fbuf[r0, pl.ds(c * L, L)]
            for c in range(4):
                abuf[a0 + 1, pl.ds(c * L, L)] = ebuf[r0, pl.ds(p0 + c * L, L)]
            for c in range(4):
                abuf[a0 + 1, pl.ds(EMB_DIM + c * L, L)] = (
                    fbuf[r1, pl.ds(c * L, L)])
            for c in range(4):
                abuf[a0 + 2, pl.ds(c * L, L)] = (
                    fbuf[r1, pl.ds(EMB_DIM + c * L, L)])
            for c in range(4):
                abuf[a0 + 2, pl.ds(EMB_DIM + c * L, L)] = (
                    ebuf[r1, pl.ds(p1 + c * L, L)])
            return _

        lax.fori_loop(0, C // 2, assemble, 0)
        pltpu.sync_copy(
            abuf, out_hbm.at[pl.ds(wid * (BPW * OUT_DIM // FEAT_DIM)
                                   + j * AROWS, AROWS)])


def kernel(indices, features, emb_table):
    idx = indices.astype(jnp.int32)
    emb2 = emb_table.reshape(NUM_NODES // 2, 2 * EMB_DIM)
    flat = _encoder(idx, features, emb2)
    return flat.reshape(BATCH, OUT_DIM)
